# baseline (speedup 1.0000x reference)
import functools

import jax
import jax.numpy as jnp
import numpy as np
from jax import lax
from jax.experimental import pallas as pl
from jax.experimental.pallas import tpu as pltpu
from jax.experimental.pallas import tpu_sc as plsc

_N, _D, _E = 50000, 256, 100000
_R = _N + _E
_NC, _NS = 2, 16
_NW = _NC * _NS
_B = 60
_NCHT = _R // _B
_T = -(-_NCHT // _NW)
_TP = _T + (_T % 2)
_U = _TP // 2

_mesh = plsc.VectorSubcoreMesh(core_axis_name="c", subcore_axis_name="s")

_slot_to_chunk = np.minimum(
    np.arange(_NW)[:, None] + np.arange(_TP)[None, :] * _NW, _NCHT - 1
)


@functools.partial(
    pl.kernel,
    out_type=jax.ShapeDtypeStruct((_R, _D), jnp.float32),
    mesh=_mesh,
    scratch_types=[
        pltpu.VMEM((_TP, 2 * _B), jnp.int32),
        pltpu.VMEM((2 * _B, _D), jnp.float32),
        pltpu.VMEM((2 * _B, _D), jnp.float32),
        pltpu.VMEM((_B, _D), jnp.float32),
        pltpu.VMEM((_B, _D), jnp.float32),
        pltpu.SemaphoreType.DMA,
        pltpu.SemaphoreType.DMA,
        pltpu.SemaphoreType.DMA,
        pltpu.SemaphoreType.DMA,
    ],
    compiler_params=pltpu.CompilerParams(use_tc_tiling_on_sc=False),
)
def _graph_pool(x_hbm, idx_hbm, out_hbm, idx_v, gb0, gb1, rb0, rb1,
                gs0, gs1, ss0, ss1):
    w = lax.axis_index("s") * _NC + lax.axis_index("c")
    gb = (gb0, gb1)
    rb = (rb0, rb1)
    gs = (gs0, gs1)
    ss = (ss0, ss1)

    pltpu.sync_copy(idx_hbm.at[w], idx_v)

    def valid(t):
        return w + t * _NW < _NCHT

    def issue_gather(t, k):
        @pl.when(valid(t))
        def _():
            pltpu.async_copy(x_hbm.at[idx_v.at[t]], gb[k], gs[k])

    def wait_gather(t, k):
        @pl.when(valid(t))
        def _():
            pltpu.make_async_copy(x_hbm.at[idx_v.at[t]], gb[k], gs[k]).wait()

    def out_slice(t):
        return out_hbm.at[pl.ds((w + t * _NW) * _B, _B)]

    def issue_scatter(t, k):
        @pl.when(valid(t))
        def _():
            pltpu.async_copy(rb[k], out_slice(t), ss[k])

    def wait_scatter(t, k):
        @pl.when((t >= 0) & valid(t))
        def _():
            pltpu.make_async_copy(rb[k], out_slice(t), ss[k]).wait()

    def compute(t, k):
        @pl.when(valid(t))
        def _():
            src, dst = gb[k], rb[k]

            def row_body(j, rc):
                for q in range(_D // 16):
                    v0 = src[2 * j, pl.ds(q * 16, 16)]
                    v1 = src[2 * j + 1, pl.ds(q * 16, 16)]
                    dst[j, pl.ds(q * 16, 16)] = (v0 + v1) * 0.5
                return rc

            lax.fori_loop(0, _B, row_body, 0, unroll=False)

    issue_gather(0, 0)
    issue_gather(1, 1)

    def step(u, carry):
        for k in range(2):
            t = 2 * u + k
            wait_gather(t, k)
            wait_scatter(t - 2, k)
            compute(t, k)
            issue_scatter(t, k)
            issue_gather(t + 2, k)
        return carry

    lax.fori_loop(0, _U, step, 0, unroll=False)

    wait_scatter(2 * _U - 2, 0)
    wait_scatter(2 * _U - 1, 1)


def kernel(inputs, pool_idx):
    ii = jnp.arange(_N, dtype=jnp.int32)
    pairs = jnp.concatenate(
        [jnp.stack([ii, ii], axis=1),
         pool_idx.reshape(_E, 2).astype(jnp.int32)], axis=0)
    blocks = pairs.reshape(_NCHT, 2 * _B)
    idx = jnp.take(blocks, jnp.asarray(_slot_to_chunk.reshape(-1)), axis=0)
    idx = idx.reshape(_NW, _TP, 2 * _B)
    return _graph_pool(inputs, idx)


# speedup vs baseline: 1.2504x; 1.2504x over previous
import functools

import jax
import jax.numpy as jnp
from jax import lax
from jax.experimental import pallas as pl
from jax.experimental.pallas import tpu as pltpu
from jax.experimental.pallas import tpu_sc as plsc

_N, _D, _E = 50000, 256, 100000
_NC, _NS = 2, 16
_NW = _NC * _NS
_B = 80
_NCHT = _E // _B
_T = -(-_NCHT // _NW)
_U = _T // 2
_CROWS = 125
_NCOPY = _N // _CROWS
_VT = -(-_NCOPY // _NW)
_CHB = 2 * _B

_mesh = plsc.VectorSubcoreMesh(core_axis_name="c", subcore_axis_name="s")


@functools.partial(
    pl.kernel,
    out_type=jax.ShapeDtypeStruct((_N + _E, _D), jnp.float32),
    mesh=_mesh,
    scratch_types=[
        pltpu.VMEM((_CHB,), jnp.int32),
        pltpu.VMEM((_CHB,), jnp.int32),
        pltpu.VMEM((_CHB, _D), jnp.float32),
        pltpu.VMEM((_CHB, _D), jnp.float32),
        pltpu.VMEM((_B, _D), jnp.float32),
        pltpu.VMEM((_B, _D), jnp.float32),
        pltpu.SemaphoreType.DMA,
        pltpu.SemaphoreType.DMA,
        pltpu.SemaphoreType.DMA,
        pltpu.SemaphoreType.DMA,
        pltpu.SemaphoreType.DMA,
        pltpu.SemaphoreType.DMA,
        pltpu.SemaphoreType.DMA,
        pltpu.SemaphoreType.DMA,
    ],
    compiler_params=pltpu.CompilerParams(use_tc_tiling_on_sc=False),
)
def _graph_pool(x_hbm, idx_hbm, out_hbm, ib0, ib1, gb0, gb1, rb0, rb1,
                is0, is1, gs0, gs1, ss0, ss1, cs0, cs1):
    w = lax.axis_index("s") * _NC + lax.axis_index("c")
    ib = (ib0, ib1)
    gb = (gb0, gb1)
    rb = (rb0, rb1)
    isem = (is0, is1)
    gsem = (gs0, gs1)
    ssem = (ss0, ss1)
    csem = (cs0, cs1)

    def valid(t):
        return w + t * _NW < _NCHT

    def idx_copy(t, k):
        return pltpu.make_async_copy(idx_hbm.at[w + t * _NW], ib[k], isem[k])

    def gather_copy(k):
        return pltpu.make_async_copy(x_hbm.at[ib[k]], gb[k], gsem[k])

    def scatter_copy(t, k):
        base = (_N + (w + t * _NW) * _B)
        return pltpu.make_async_copy(rb[k], out_hbm.at[pl.ds(base, _B)],
                                     ssem[k])

    def issue_idx(t, k):
        @pl.when(valid(t))
        def _():
            idx_copy(t, k).start()

    def wait_idx(t, k):
        @pl.when(valid(t))
        def _():
            idx_copy(t, k).wait()

    def issue_gather(t, k):
        @pl.when(valid(t))
        def _():
            gather_copy(k).start()

    def wait_gather(t, k):
        @pl.when(valid(t))
        def _():
            gather_copy(k).wait()

    def issue_scatter(t, k):
        @pl.when(valid(t))
        def _():
            scatter_copy(t, k).start()

    def wait_scatter(t, k):
        @pl.when((t >= 0) & valid(t))
        def _():
            scatter_copy(t, k).wait()

    def compute(t, k):
        @pl.when(valid(t))
        def _():
            src, dst = gb[k], rb[k]

            def row_body(j, rc):
                for q in range(_D // 16):
                    v0 = src[2 * j, pl.ds(q * 16, 16)]
                    v1 = src[2 * j + 1, pl.ds(q * 16, 16)]
                    dst[j, pl.ds(q * 16, 16)] = (v0 + v1) * 0.5
                return rc

            lax.fori_loop(0, _B, row_body, 0, unroll=False)

    issue_idx(0, 0)
    issue_idx(1, 1)
    wait_idx(0, 0)
    issue_gather(0, 0)

    def step(u, carry):
        for k in range(2):
            t = 2 * u + k
            wait_gather(t, k)
            issue_idx(t + 2, k)
            wait_idx(t + 1, 1 - k)
            issue_gather(t + 1, 1 - k)
            wait_scatter(t - 2, k)
            compute(t, k)
            issue_scatter(t, k)
        return carry

    lax.fori_loop(0, _U, step, 0, unroll=False)

    wait_scatter(2 * _U - 2, 0)
    wait_scatter(2 * _U - 1, 1)

    def cvalid(v):
        return w + v * _NW < _NCOPY

    def read_copy(v, k):
        r0 = (w + v * _NW) * _CROWS
        return pltpu.make_async_copy(x_hbm.at[pl.ds(r0, _CROWS)],
                                     gb[k].at[pl.ds(0, _CROWS)], csem[k])

    def write_copy(v, k):
        r0 = (w + v * _NW) * _CROWS
        return pltpu.make_async_copy(gb[k].at[pl.ds(0, _CROWS)],
                                     out_hbm.at[pl.ds(r0, _CROWS)], ssem[k])

    def issue_read(v, k):
        @pl.when(cvalid(v))
        def _():
            read_copy(v, k).start()

    def wait_read(v, k):
        @pl.when(cvalid(v))
        def _():
            read_copy(v, k).wait()

    def issue_write(v, k):
        @pl.when(cvalid(v))
        def _():
            write_copy(v, k).start()

    def wait_write(v, k):
        @pl.when((v >= 0) & cvalid(v))
        def _():
            write_copy(v, k).wait()

    issue_read(0, 0)

    def cstep(u, carry):
        for k in range(2):
            v = 2 * u + k
            wait_read(v, k)
            issue_write(v, k)
            wait_write(v - 1, 1 - k)
            issue_read(v + 1, 1 - k)
        return carry

    lax.fori_loop(0, (_VT + 1) // 2, cstep, 0, unroll=False)


def kernel(inputs, pool_idx):
    idx = pool_idx.reshape(_NCHT, _CHB).astype(jnp.int32)
    return _graph_pool(inputs, idx)


# speedup vs baseline: 1.2579x; 1.0060x over previous
import functools

import jax
import jax.numpy as jnp
from jax import lax
from jax.experimental import pallas as pl
from jax.experimental.pallas import tpu as pltpu
from jax.experimental.pallas import tpu_sc as plsc

_N, _D, _E = 50000, 256, 100000
_NC, _NS = 2, 16
_NW = _NC * _NS
_B = 40
_NCHT = _E // _B
_T = 80
_CHB = 2 * _B
_CROWS = 80
_NCOPY = _N // _CROWS
_VT = 20

_mesh = plsc.VectorSubcoreMesh(core_axis_name="c", subcore_axis_name="s")


@functools.partial(
    pl.kernel,
    out_type=jax.ShapeDtypeStruct((_N + _E, _D), jnp.float32),
    mesh=_mesh,
    scratch_types=[
        [pltpu.VMEM((_CHB,), jnp.int32) for _ in range(4)],
        [pltpu.VMEM((_CHB, _D), jnp.float32) for _ in range(4)],
        [pltpu.VMEM((_B, _D), jnp.float32) for _ in range(2)],
        [pltpu.SemaphoreType.DMA for _ in range(4)],
        [pltpu.SemaphoreType.DMA for _ in range(4)],
        [pltpu.SemaphoreType.DMA for _ in range(2)],
    ],
    compiler_params=pltpu.CompilerParams(use_tc_tiling_on_sc=False),
)
def _graph_pool(x_hbm, idx_hbm, out_hbm, ib, gb, rb, isem, gsem, ssem):
    w = lax.axis_index("s") * _NC + lax.axis_index("c")

    def valid(t):
        return w + t * _NW < _NCHT

    def idx_copy(t, k):
        return pltpu.make_async_copy(idx_hbm.at[w + t * _NW], ib[k], isem[k])

    def gather_copy(k):
        return pltpu.make_async_copy(x_hbm.at[ib[k]], gb[k], gsem[k])

    def scatter_copy(t, k2):
        base = _N + (w + t * _NW) * _B
        return pltpu.make_async_copy(rb[k2], out_hbm.at[pl.ds(base, _B)],
                                     ssem[k2])

    def issue_idx(t, k):
        @pl.when(valid(t))
        def _():
            idx_copy(t, k).start()

    def wait_idx(t, k):
        @pl.when(valid(t))
        def _():
            idx_copy(t, k).wait()

    def issue_gather(t, k):
        @pl.when(valid(t))
        def _():
            gather_copy(k).start()

    def wait_gather(t, k):
        @pl.when(valid(t))
        def _():
            gather_copy(k).wait()

    def issue_scatter(t, k2):
        @pl.when(valid(t))
        def _():
            scatter_copy(t, k2).start()

    def wait_scatter(t, k2):
        @pl.when((t >= 0) & valid(t))
        def _():
            scatter_copy(t, k2).wait()

    def compute(t, k, k2):
        @pl.when(valid(t))
        def _():
            src, dst = gb[k], rb[k2]

            def row_body(j, rc):
                for q in range(_D // 16):
                    v0 = src[2 * j, pl.ds(q * 16, 16)]
                    v1 = src[2 * j + 1, pl.ds(q * 16, 16)]
                    dst[j, pl.ds(q * 16, 16)] = (v0 + v1) * 0.5
                return rc

            lax.fori_loop(0, _B, row_body, 0, unroll=False)

    for t in range(3):
        issue_idx(t, t)
    for t in range(3):
        wait_idx(t, t)
        issue_gather(t, t)
    issue_idx(3, 3)

    def step(u, carry):
        for k in range(4):
            t = 4 * u + k
            k3 = (k + 3) % 4
            k2 = k % 2
            wait_gather(t, k)
            issue_idx(t + 4, k)
            wait_idx(t + 3, k3)
            issue_gather(t + 3, k3)
            wait_scatter(t - 2, k2)
            compute(t, k, k2)
            issue_scatter(t, k2)
        return carry

    lax.fori_loop(0, _T // 4, step, 0, unroll=False)

    wait_scatter(_T - 2, 0)
    wait_scatter(_T - 1, 1)

    def cvalid(v):
        return w + v * _NW < _NCOPY

    def read_copy(v, k):
        r0 = (w + v * _NW) * _CROWS
        return pltpu.make_async_copy(x_hbm.at[pl.ds(r0, _CROWS)],
                                     gb[k], gsem[k])

    def write_copy(v, k):
        r0 = (w + v * _NW) * _CROWS
        return pltpu.make_async_copy(gb[k], out_hbm.at[pl.ds(r0, _CROWS)],
                                     ssem[k % 2])

    def issue_read(v, k):
        @pl.when(cvalid(v))
        def _():
            read_copy(v, k).start()

    def wait_read(v, k):
        @pl.when(cvalid(v))
        def _():
            read_copy(v, k).wait()

    def issue_write(v, k):
        @pl.when(cvalid(v))
        def _():
            write_copy(v, k).start()

    def wait_write(v, k):
        @pl.when((v >= 0) & cvalid(v))
        def _():
            write_copy(v, k).wait()

    for v in range(3):
        issue_read(v, v)

    def cstep(u, carry):
        for k in range(4):
            v = 4 * u + k
            wait_read(v, k)
            issue_write(v, k)
            wait_write(v - 1, (k - 1) % 4)
            issue_read(v + 3, (k + 3) % 4)
        return carry

    lax.fori_loop(0, _VT // 4, cstep, 0, unroll=False)

    wait_write(_VT - 1, (_VT - 1) % 4)


def kernel(inputs, pool_idx):
    idx = pool_idx.reshape(_NCHT, _CHB).astype(jnp.int32)
    return _graph_pool(inputs, idx)
